# Initial kernel scaffold; baseline (speedup 1.0000x reference)
import jax
import jax.numpy as jnp
from jax.experimental import pallas as pl


def kernel(router_logits, mask):
    raise NotImplementedError("write your pallas kernel here")


# speedup vs baseline: 4.2091x; 4.2091x over previous
import functools

import jax
import jax.numpy as jnp
from jax import lax
from jax.experimental import pallas as pl
from jax.experimental.pallas import tpu as pltpu
from jax.experimental.pallas import tpu_sc as plsc

NUM_LAYERS = 24
TOK_PER_LAYER = 16384
E = 64
K = 8
N_TOK = NUM_LAYERS * TOK_PER_LAYER
NUM_WORKERS = 32
TOK_PER_TILE = N_TOK // NUM_WORKERS
T_CHUNK = 512
N_CHUNKS = TOK_PER_TILE // T_CHUNK


def _tile_body(logits_hbm, mask_hbm, out_hbm, lbuf, mbuf, obuf):
    wid = lax.axis_index("s") * 2 + lax.axis_index("c")
    tile_start = wid * TOK_PER_TILE

    pltpu.sync_copy(mask_hbm, mbuf)

    iota = lax.iota(jnp.int32, 16)
    idx_lo8 = iota & 7
    sel_lo = iota < 8
    tau_idx = jnp.full((16,), 7, jnp.int32)

    def lane_gather(x, idx):
        return lax.gather(
            x,
            idx[:, None],
            dimension_numbers=lax.GatherDimensionNumbers(
                offset_dims=(), collapsed_slice_dims=(0,),
                start_index_map=(0,)),
            slice_sizes=(1,),
            mode=lax.GatherScatterMode.PROMISE_IN_BOUNDS,
        )

    def sortd(x):
        res = plsc.sort_key_val(x, x, descending=True)
        return res[-1] if isinstance(res, (tuple, list)) else res

    def merge(a, b):
        return sortd(jnp.where(sel_lo, a, lane_gather(b, idx_lo8)))

    def lane_sum_splat(x):
        for sh in (8, 4, 2, 1):
            x = x + lane_gather(x, iota ^ sh)
        return x

    def token_body(i, carry):
        c0, c1, c2, c3, p0, p1, p2, p3, tok = carry
        base = i * E
        v0 = lbuf[pl.ds(base, 16)]
        v1 = lbuf[pl.ds(base + 16, 16)]
        v2 = lbuf[pl.ds(base + 32, 16)]
        v3 = lbuf[pl.ds(base + 48, 16)]

        e0 = jnp.exp(v0)
        e1 = jnp.exp(v1)
        e2 = jnp.exp(v2)
        e3 = jnp.exp(v3)
        denom = lane_sum_splat((e0 + e1) + (e2 + e3))

        s0 = sortd(v0)
        s1 = sortd(v1)
        s2 = sortd(v2)
        s3 = sortd(v3)
        f = merge(merge(s0, s1), merge(s2, s3))
        tau = lane_gather(f, tau_idx)

        j = lax.rem(tok, TOK_PER_LAYER)
        lane = lax.rem(tok, 16)
        mv = mbuf[pl.ds(j - lane, 16)]
        m = lane_gather(mv, jnp.full((16,), lane, jnp.int32))
        w = m / denom
        zero = jnp.zeros((16,), jnp.float32)
        return (
            c0 + jnp.where(v0 >= tau, m, zero),
            c1 + jnp.where(v1 >= tau, m, zero),
            c2 + jnp.where(v2 >= tau, m, zero),
            c3 + jnp.where(v3 >= tau, m, zero),
            p0 + e0 * w,
            p1 + e1 * w,
            p2 + e2 * w,
            p3 + e3 * w,
            tok + 1,
        )

    def chunk_body(c, carry):
        chunk_start = tile_start + c * T_CHUNK
        pltpu.sync_copy(
            logits_hbm.at[pl.ds(chunk_start * E, T_CHUNK * E)], lbuf
        )
        return lax.fori_loop(0, T_CHUNK, token_body, carry)

    zeros = jnp.zeros((16,), jnp.float32)
    init = (zeros,) * 8 + (tile_start,)
    res = lax.fori_loop(0, N_CHUNKS, chunk_body, init)

    for j in range(4):
        obuf[pl.ds(j * 16, 16)] = res[j]
        obuf[pl.ds(64 + j * 16, 16)] = res[4 + j]
    pltpu.sync_copy(obuf, out_hbm.at[pl.ds(wid * 128, 128)])


@jax.jit
def kernel(router_logits, mask):
    logits_flat = router_logits.reshape(-1)
    mask_flat = mask.reshape(-1).astype(jnp.float32)

    mesh = plsc.VectorSubcoreMesh(core_axis_name="c", subcore_axis_name="s")
    partials = pl.kernel(
        _tile_body,
        mesh=mesh,
        compiler_params=pltpu.CompilerParams(needs_layout_passes=False),
        out_type=jax.ShapeDtypeStruct((NUM_WORKERS * 128,), jnp.float32),
        scratch_types=[
            pltpu.VMEM((T_CHUNK * E,), jnp.float32),
            pltpu.VMEM((TOK_PER_LAYER,), jnp.float32),
            pltpu.VMEM((128,), jnp.float32),
        ],
    )(logits_flat, mask_flat)

    parts = partials.reshape(NUM_WORKERS, 2, 64)
    C = parts[:, 0, :].sum(axis=0)
    P = parts[:, 1, :].sum(axis=0)
    S = jnp.sum(mask_flat) * NUM_LAYERS
    return jnp.float32(E) * jnp.dot(C, P) / (S * S)
